# Initial kernel scaffold; baseline (speedup 1.0000x reference)
import jax
import jax.numpy as jnp
from jax.experimental import pallas as pl


def kernel(tgt, memory, pos, query_pos, pos_centers, sa_Wq, sa_bq, sa_Wk, sa_bk, sa_Wv, sa_bv, sa_Wo, sa_bo, ca_Wq, ca_bq, ca_Wk, ca_bk, ca_Wv, ca_bv, ca_Wo, ca_bo, ffn_W1, ffn_b1, ffn_W2, ffn_b2, ln1_g, ln1_b, ln2_g, ln2_b, ln3_g, ln3_b, norm_g, norm_b):
    raise NotImplementedError("write your pallas kernel here")


# speedup vs baseline: 2.2006x; 2.2006x over previous
import jax
import jax.numpy as jnp
from jax.experimental import pallas as pl

L = 2
D = 256
H = 8
FF = 2048
NQ = 300
B = 2
S = 4096
TOPK = 100
DH = D // H
NQP = 304
CH = 8
NCH = NQP // CH
_SCALE = 1.0 / (DH ** 0.5)
_F32 = jnp.float32


def _layer_norm(x, g, b, eps=1e-5):
    m = jnp.mean(x, axis=-1, keepdims=True)
    v = jnp.mean((x - m) ** 2, axis=-1, keepdims=True)
    return (x - m) / jnp.sqrt(v + eps) * g + b


def _dot(a, b):
    return jnp.dot(a, b, preferred_element_type=_F32)


def _dot_t(a, b):
    return jax.lax.dot_general(a, b, (((1,), (1,)), ((), ())),
                               preferred_element_type=_F32)


def _giou_kernel(pcr_ref, pcc_ref, s_ref):
    def boxes_r():
        cx = jax.nn.sigmoid(pcr_ref[0, :, 0:1])
        cy = jax.nn.sigmoid(pcr_ref[0, :, 1:2])
        w = jax.nn.sigmoid(pcr_ref[0, :, 2:3])
        h = jax.nn.sigmoid(pcr_ref[0, :, 3:4])
        return cx - 0.5 * w, cy - 0.5 * h, cx + 0.5 * w, cy + 0.5 * h

    def boxes_c():
        cx = jax.nn.sigmoid(pcc_ref[0, 0:1, :])
        cy = jax.nn.sigmoid(pcc_ref[0, 1:2, :])
        w = jax.nn.sigmoid(pcc_ref[0, 2:3, :])
        h = jax.nn.sigmoid(pcc_ref[0, 3:4, :])
        return cx - 0.5 * w, cy - 0.5 * h, cx + 0.5 * w, cy + 0.5 * h

    x0r, y0r, x1r, y1r = boxes_r()
    x0c, y0c, x1c, y1c = boxes_c()
    a_r = (x1r - x0r) * (y1r - y0r)
    a_c = (x1c - x0c) * (y1c - y0c)
    wi = jnp.clip(jnp.minimum(x1r, x1c) - jnp.maximum(x0r, x0c), 0.0)
    hi = jnp.clip(jnp.minimum(y1r, y1c) - jnp.maximum(y0r, y0c), 0.0)
    inter = wi * hi
    union = a_r + a_c - inter
    iou = inter / union
    wc = jnp.clip(jnp.maximum(x1r, x1c) - jnp.minimum(x0r, x0c), 0.0)
    hc = jnp.clip(jnp.maximum(y1r, y1c) - jnp.minimum(y0r, y0c), 0.0)
    area = wc * hc
    score = 1.0 - (iou - (area - union) / area)
    col = jax.lax.broadcasted_iota(jnp.int32, (NQP, NQP), 1)
    s_ref[0] = jnp.where(col >= NQ, 1e9, score)


def _rank_kernel(s_ref, mask_ref):
    c = pl.program_id(1)
    sc = s_ref[0, pl.ds(c * CH, CH), :]
    lt = (sc[:, None, :] < sc[:, :, None]).astype(_F32)
    kj = (jax.lax.broadcasted_iota(jnp.int32, (NQP, NQP), 1)
          < jax.lax.broadcasted_iota(jnp.int32, (NQP, NQP), 0))
    eq = ((sc[:, None, :] == sc[:, :, None]) & kj[None, :, :]).astype(_F32)
    rank = jnp.sum(lt, axis=2) + jnp.sum(eq, axis=2)
    mask_ref[0] = (rank < float(TOPK)).astype(_F32)


def _kv_kernel(mem_ref, p_ref, wk_ref, bk_ref, wv_ref, bv_ref, k_ref, v_ref):
    mem = mem_ref[0]
    k_ref[0] = _dot(mem + p_ref[0], wk_ref[...]) + bk_ref[...]
    v_ref[0] = _dot(mem, wv_ref[...]) + bv_ref[...]


def _sa_kernel(x_ref, qp_ref, mask_ref, wq_ref, bq_ref, wk_ref, bk_ref,
               wv_ref, bv_ref, wo_ref, bo_ref, g_ref, b_ref, o_ref):
    x = x_ref[0]
    q = x + qp_ref[0]
    m = mask_ref[0]
    qa = _dot(q, wq_ref[...]) + bq_ref[...]
    ka = _dot(q, wk_ref[...]) + bk_ref[...]
    va = _dot(x, wv_ref[...]) + bv_ref[...]
    outs = []
    for h in range(H):
        sl = slice(h * DH, (h + 1) * DH)
        s = _dot_t(qa[:, sl], ka[:, sl]) * _SCALE
        s = jnp.where(m > 0, s, -1e9)
        s = s - jnp.max(s, axis=-1, keepdims=True)
        e = jnp.exp(s)
        pr = e / jnp.sum(e, axis=-1, keepdims=True)
        outs.append(_dot(pr, va[:, sl]))
    o = _dot(jnp.concatenate(outs, axis=1), wo_ref[...]) + bo_ref[...]
    o_ref[0] = _layer_norm(o + x, g_ref[...], b_ref[...])


def _make_ca_kernel(with_attn):
    def _ca_kernel(x_ref, qp_ref, k_ref, v_ref, wq_ref, bq_ref, wo_ref,
                   bo_ref, g_ref, b_ref, o_ref, *maybe_attn):
        x = x_ref[0]
        qa = _dot(x + qp_ref[0], wq_ref[...]) + bq_ref[...]
        ka = k_ref[0]
        va = v_ref[0]
        outs = []
        attn_acc = None
        for h in range(H):
            sl = slice(h * DH, (h + 1) * DH)
            s = _dot_t(qa[:, sl], ka[:, sl]) * _SCALE
            s = s - jnp.max(s, axis=-1, keepdims=True)
            e = jnp.exp(s)
            pr = e / jnp.sum(e, axis=-1, keepdims=True)
            if with_attn:
                attn_acc = pr if attn_acc is None else attn_acc + pr
            outs.append(_dot(pr, va[:, sl]))
        o = _dot(jnp.concatenate(outs, axis=1), wo_ref[...]) + bo_ref[...]
        o_ref[0] = _layer_norm(o + x, g_ref[...], b_ref[...])
        if with_attn:
            maybe_attn[0][0] = attn_acc * (1.0 / H)
    return _ca_kernel


def _make_ffn_kernel(final):
    def _ffn_kernel(x_ref, w1_ref, b1_ref, w2_ref, b2_ref, g_ref, b_ref,
                    ng_ref, nb_ref, o_ref):
        x = x_ref[0]
        hmid = jnp.maximum(_dot(x, w1_ref[...]) + b1_ref[...], 0.0)
        y = _dot(hmid, w2_ref[...]) + b2_ref[...] + x
        y = _layer_norm(y, g_ref[...], b_ref[...])
        if final:
            y = _layer_norm(y, ng_ref[...], nb_ref[...])
        o_ref[0] = y
    return _ffn_kernel


def _bspec(shape, index_map):
    return pl.BlockSpec(shape, index_map)


def _full2d(arr):
    return pl.BlockSpec(arr.shape, lambda *_: (0,) * arr.ndim)


def _row(v):
    return v.reshape(1, -1)


def kernel(tgt, memory, pos, query_pos, pos_centers, sa_Wq, sa_bq, sa_Wk,
           sa_bk, sa_Wv, sa_bv, sa_Wo, sa_bo, ca_Wq, ca_bq, ca_Wk, ca_bk,
           ca_Wv, ca_bv, ca_Wo, ca_bo, ffn_W1, ffn_b1, ffn_W2, ffn_b2,
           ln1_g, ln1_b, ln2_g, ln2_b, ln3_g, ln3_b, norm_g, norm_b):
    f32 = _F32
    pad_q = ((0, 0), (0, NQP - NQ), (0, 0))
    xb = jnp.pad(tgt.transpose(1, 0, 2), pad_q)
    qpb = jnp.pad(query_pos.transpose(1, 0, 2), pad_q)
    memb = memory.transpose(1, 0, 2)
    pb = pos.transpose(1, 0, 2)
    pcr = jnp.pad(pos_centers.transpose(1, 0, 2), pad_q)
    pcc = jnp.pad(pos_centers.transpose(1, 2, 0),
                  ((0, 0), (0, 0), (0, NQP - NQ)))

    scores = pl.pallas_call(
        _giou_kernel,
        grid=(B,),
        in_specs=[_bspec((1, NQP, 4), lambda b: (b, 0, 0)),
                  _bspec((1, 4, NQP), lambda b: (b, 0, 0))],
        out_specs=_bspec((1, NQP, NQP), lambda b: (b, 0, 0)),
        out_shape=jax.ShapeDtypeStruct((B, NQP, NQP), f32),
    )(pcr, pcc)

    mask = pl.pallas_call(
        _rank_kernel,
        grid=(B, NCH),
        in_specs=[_bspec((1, NQP, NQP), lambda b, c: (b, 0, 0))],
        out_specs=_bspec((1, CH, NQP), lambda b, c: (b, c, 0)),
        out_shape=jax.ShapeDtypeStruct((B, NQP, NQP), f32),
    )(scores)

    wk_cat = jnp.concatenate([ca_Wk[0], ca_Wk[1]], axis=1)
    bk_cat = _row(jnp.concatenate([ca_bk[0], ca_bk[1]]))
    wv_cat = jnp.concatenate([ca_Wv[0], ca_Wv[1]], axis=1)
    bv_cat = _row(jnp.concatenate([ca_bv[0], ca_bv[1]]))
    SCNK = 4
    kall, vall = pl.pallas_call(
        _kv_kernel,
        grid=(B, SCNK),
        in_specs=[_bspec((1, S // SCNK, D), lambda b, i: (b, i, 0)),
                  _bspec((1, S // SCNK, D), lambda b, i: (b, i, 0)),
                  _bspec((D, 2 * D), lambda b, i: (0, 0)),
                  _bspec((1, 2 * D), lambda b, i: (0, 0)),
                  _bspec((D, 2 * D), lambda b, i: (0, 0)),
                  _bspec((1, 2 * D), lambda b, i: (0, 0))],
        out_specs=[_bspec((1, S // SCNK, 2 * D), lambda b, i: (b, i, 0)),
                   _bspec((1, S // SCNK, 2 * D), lambda b, i: (b, i, 0))],
        out_shape=[jax.ShapeDtypeStruct((B, S, 2 * D), f32),
                   jax.ShapeDtypeStruct((B, S, 2 * D), f32)],
    )(memb, pb, wk_cat, bk_cat, wv_cat, bv_cat)

    x = xb
    dec_attn = None
    for layer in range(L):
        x = pl.pallas_call(
            _sa_kernel,
            grid=(B,),
            in_specs=[_bspec((1, NQP, D), lambda b: (b, 0, 0)),
                      _bspec((1, NQP, D), lambda b: (b, 0, 0)),
                      _bspec((1, NQP, NQP), lambda b: (b, 0, 0)),
                      _bspec((D, D), lambda b: (0, 0)),
                      _bspec((1, D), lambda b: (0, 0)),
                      _bspec((D, D), lambda b: (0, 0)),
                      _bspec((1, D), lambda b: (0, 0)),
                      _bspec((D, D), lambda b: (0, 0)),
                      _bspec((1, D), lambda b: (0, 0)),
                      _bspec((D, D), lambda b: (0, 0)),
                      _bspec((1, D), lambda b: (0, 0)),
                      _bspec((1, D), lambda b: (0, 0)),
                      _bspec((1, D), lambda b: (0, 0))],
            out_specs=_bspec((1, NQP, D), lambda b: (b, 0, 0)),
            out_shape=jax.ShapeDtypeStruct((B, NQP, D), f32),
        )(x, qpb, mask, sa_Wq[layer], _row(sa_bq[layer]), sa_Wk[layer],
          _row(sa_bk[layer]), sa_Wv[layer], _row(sa_bv[layer]), sa_Wo[layer],
          _row(sa_bo[layer]), _row(ln1_g[layer]), _row(ln1_b[layer]))

        with_attn = layer == L - 1
        kv_spec = _bspec((1, S, D), lambda b, layer=layer: (b, 0, layer))
        ca_in_specs = [_bspec((1, NQP, D), lambda b: (b, 0, 0)),
                       _bspec((1, NQP, D), lambda b: (b, 0, 0)),
                       kv_spec, kv_spec,
                       _bspec((D, D), lambda b: (0, 0)),
                       _bspec((1, D), lambda b: (0, 0)),
                       _bspec((D, D), lambda b: (0, 0)),
                       _bspec((1, D), lambda b: (0, 0)),
                       _bspec((1, D), lambda b: (0, 0)),
                       _bspec((1, D), lambda b: (0, 0))]
        ca_out_specs = [_bspec((1, NQP, D), lambda b: (b, 0, 0))]
        ca_out_shape = [jax.ShapeDtypeStruct((B, NQP, D), f32)]
        if with_attn:
            ca_out_specs.append(_bspec((1, NQP, S), lambda b: (b, 0, 0)))
            ca_out_shape.append(jax.ShapeDtypeStruct((B, NQP, S), f32))
        ca_res = pl.pallas_call(
            _make_ca_kernel(with_attn),
            grid=(B,),
            in_specs=ca_in_specs,
            out_specs=ca_out_specs,
            out_shape=ca_out_shape,
        )(x, qpb, kall, vall, ca_Wq[layer], _row(ca_bq[layer]),
          ca_Wo[layer], _row(ca_bo[layer]), _row(ln2_g[layer]),
          _row(ln2_b[layer]))
        if with_attn:
            x, attn_full = ca_res
            dec_attn = attn_full[:, :NQ, :]
        else:
            x, = ca_res

        x = pl.pallas_call(
            _make_ffn_kernel(final=(layer == L - 1)),
            grid=(B,),
            in_specs=[_bspec((1, NQP, D), lambda b: (b, 0, 0)),
                      _bspec((D, FF), lambda b: (0, 0)),
                      _bspec((1, FF), lambda b: (0, 0)),
                      _bspec((FF, D), lambda b: (0, 0)),
                      _bspec((1, D), lambda b: (0, 0)),
                      _bspec((1, D), lambda b: (0, 0)),
                      _bspec((1, D), lambda b: (0, 0)),
                      _bspec((1, D), lambda b: (0, 0)),
                      _bspec((1, D), lambda b: (0, 0))],
            out_specs=_bspec((1, NQP, D), lambda b: (b, 0, 0)),
            out_shape=jax.ShapeDtypeStruct((B, NQP, D), f32),
        )(x, ffn_W1[layer], _row(ffn_b1[layer]), ffn_W2[layer],
          _row(ffn_b2[layer]), _row(ln3_g[layer]), _row(ln3_b[layer]),
          _row(norm_g), _row(norm_b))

    out = x[:, :NQ, :].transpose(1, 0, 2)
    return (out, pos_centers, dec_attn)
